# baseline (speedup 1.0000x reference)
import functools

import jax
import jax.numpy as jnp
from jax import lax
from jax.experimental import pallas as pl
from jax.experimental.pallas import tpu as pltpu
from jax.experimental.pallas import tpu_sc as plsc

_N = 10000
_E = 320000
_D = 128
_G = 16

_NP = 10240
_EP = 327680
_NC = 2
_NS = 16
_L = 16
_EPT = _EP // (_NC * _NS)
_CH = 128
_NG = _EPT // _CH
_RPT = _NP // _NS

_BN = 512
_NB = _NP // _BN


def _make_sc_agg(d_row, with_deg, ng0=_NG, ng1=_NG):
  assert ng0 % 4 == 0 and ng1 % 4 == 0 and 16 * (ng0 + ng1) * _CH == _EP
  out_type = [jax.ShapeDtypeStruct((_NC, _NP, d_row), jnp.float32)]
  scratch = [pltpu.VMEM_SHARED((_NP, d_row), jnp.float32)]
  if with_deg:
    out_type.append(jax.ShapeDtypeStruct((_NC, _NS, _NP), jnp.float32))
    scratch.append(pltpu.VMEM((_NP,), jnp.float32))
  scratch += [
      [pltpu.VMEM((_CH,), jnp.int32) for _ in range(4)],
      [pltpu.VMEM((_CH,), jnp.int32) for _ in range(4)],
      [pltpu.VMEM((_CH, d_row), jnp.float32) for _ in range(2)],
      [pltpu.SemaphoreType.DMA for _ in range(4)],
      [pltpu.SemaphoreType.DMA for _ in range(2)],
      [pltpu.SemaphoreType.DMA for _ in range(2)],
  ]

  def body(table_hbm, src_hbm, dst_hbm, *rest):
    if with_deg:
      (parts_out, deg_out, acc, deg_loc, sidx, didx, rows, sem_ix, sem_g,
       sem_s) = rest
    else:
      parts_out, acc, sidx, didx, rows, sem_ix, sem_g, sem_s = rest
      deg_out = deg_loc = None
    c = lax.axis_index("c")
    s = lax.axis_index("s")
    z16 = jnp.zeros((_L,), jnp.float32)
    npr = d_row // _L

    def zfill(i, carry):
      rows[0][i // npr, pl.ds((i % npr) * _L, _L)] = z16
      return carry

    lax.fori_loop(0, _CH * npr, zfill, 0)
    if with_deg:

      def dzfill(i, carry):
        deg_loc[pl.ds(i * _L, _L)] = z16
        return carry

      lax.fori_loop(0, _NP // _L, dzfill, 0)

    base = s * _RPT
    for j in range(_RPT // 128):
      pltpu.sync_copy(rows[0], acc.at[pl.ds(base + j * 128, 128), :])
    plsc.subcore_barrier()


    def issue_idx(ebase, g, slot):
      off = ebase + g * _CH
      pltpu.async_copy(src_hbm.at[pl.ds(off, _CH)], sidx[slot], sem_ix[slot])
      pltpu.async_copy(dst_hbm.at[pl.ds(off, _CH)], didx[slot], sem_ix[slot])

    def wait_idx(slot):
      pltpu.make_async_copy(src_hbm.at[pl.ds(0, _CH)], sidx[slot],
                            sem_ix[slot]).wait()
      pltpu.make_async_copy(dst_hbm.at[pl.ds(0, _CH)], didx[slot],
                            sem_ix[slot]).wait()

    def issue_gather(slot4, slot2):
      pltpu.async_copy(table_hbm.at[sidx[slot4]], rows[slot2], sem_g[slot2])

    def wait_gather(slot4, slot2):
      pltpu.make_async_copy(table_hbm.at[sidx[slot4]], rows[slot2],
                            sem_g[slot2]).wait()

    def issue_scatter(slot4, slot2):
      pltpu.async_copy(rows[slot2], acc.at[didx[slot4]], sem_s[slot2],
                       add=True)

    def wait_scatter(slot4, slot2):
      pltpu.make_async_copy(rows[slot2], acc.at[didx[slot4]],
                            sem_s[slot2]).wait()

    def run_edges(ebase, ng):
      issue_idx(ebase, 0, 0)
      issue_idx(ebase, 1, 1)
      issue_idx(ebase, 2, 2)
      wait_idx(0)
      issue_gather(0, 0)

      def pipe(i, carry):
        for b in range(4):
          b2 = b % 2
          nb2 = (b + 1) % 2
          nb4 = (b + 1) % 4
          wait_gather(b, b2)
          if with_deg:
            for k in range(_CH // _L):
              idx16 = didx[b][pl.ds(k * _L, _L)]
              cnt, lastm = plsc.scan_count(idx16)
              plsc.addupdate_scatter(deg_loc, [idx16],
                                     cnt.astype(jnp.float32), mask=lastm)
          issue_scatter(b, b2)
          if b == 0:
            @pl.when(i >= 1)
            def _():
              wait_scatter(3, nb2)
          else:
            wait_scatter(b - 1, nb2)
          if b < 3:
            wait_idx(nb4)
            issue_gather(nb4, nb2)
            lim = (ng - b) // 4
            @pl.when(i < lim)
            def _():
              issue_idx(ebase, 4 * i + b + 3, (b + 3) % 4)
          else:
            @pl.when(i < ng // 4 - 1)
            def _():
              wait_idx(0)
              issue_gather(0, nb2)
              issue_idx(ebase, 4 * i + 6, 2)
        return carry

      lax.fori_loop(0, ng // 4, pipe, 0)
      wait_scatter(3, 1)

    if ng0 == ng1:
      run_edges((c * _NS + s) * (ng0 * _CH), ng0)
    else:
      @pl.when(c == 0)
      def _():
        run_edges(s * (ng0 * _CH), ng0)

      @pl.when(c == 1)
      def _():
        run_edges(_NS * ng0 * _CH + s * (ng1 * _CH), ng1)
    if with_deg:
      pltpu.sync_copy(deg_loc, deg_out.at[c, s])
    plsc.subcore_barrier()

    for j in range(_RPT // 128):
      r0 = base + j * 128
      pltpu.sync_copy(acc.at[pl.ds(r0, 128), :],
                      parts_out.at[c, pl.ds(r0, 128), :])

  mesh = plsc.VectorSubcoreMesh(core_axis_name="c", subcore_axis_name="s",
                                num_cores=_NC, num_subcores=_NS)
  return pl.kernel(body, out_type=out_type, mesh=mesh,
                   scratch_types=scratch,
                   compiler_params=pltpu.CompilerParams(
                       needs_layout_passes=False))


_NG0 = 152
_NG1 = 8

_sc_agg_deg = _make_sc_agg(_D, True, _NG0, _NG1)
_sc_agg = _make_sc_agg(_D, False, _NG0, _NG1)
_sc_agg_pol = _make_sc_agg(_D, False, _NG0, _NG1)


def _tc1_body(parts, deg, x, w1l, w1r, b1, h1o, recipo):
  p = parts[...]
  d = jnp.sum(deg[...], axis=0)
  r_row = 1.0 / jnp.maximum(d, 1.0)
  ii = lax.broadcasted_iota(jnp.int32, (_BN, _BN), 0)
  jj = lax.broadcasted_iota(jnp.int32, (_BN, _BN), 1)
  diag_r = jnp.where(ii == jj, jnp.broadcast_to(r_row, (_BN, _BN)), 0.0)
  mean = jnp.dot(diag_r, p[0] + p[1], preferred_element_type=jnp.float32)
  r_col = jnp.sum(diag_r, axis=1, keepdims=True)
  h = (jnp.dot(mean, w1l[...], preferred_element_type=jnp.float32)
       + jnp.dot(x[...], w1r[...], preferred_element_type=jnp.float32)
       + b1[...])
  h1o[...] = jnp.maximum(h, 0.0)
  recipo[...] = jnp.broadcast_to(r_col, (_BN, _L))


def _tc1(parts, deg, x, w1l, w1r, b1):
  return pl.pallas_call(
      _tc1_body,
      grid=(_NB,),
      in_specs=[
          pl.BlockSpec((_NC, _BN, _D), lambda i: (0, i, 0)),
          pl.BlockSpec((_NC * _NS, 1, _BN), lambda i: (0, 0, i)),
          pl.BlockSpec((_BN, _D), lambda i: (i, 0)),
          pl.BlockSpec((_D, _D), lambda i: (0, 0)),
          pl.BlockSpec((_D, _D), lambda i: (0, 0)),
          pl.BlockSpec((1, _D), lambda i: (0, 0)),
      ],
      out_specs=[
          pl.BlockSpec((_BN, _D), lambda i: (i, 0)),
          pl.BlockSpec((_BN, _L), lambda i: (i, 0)),
      ],
      out_shape=[
          jax.ShapeDtypeStruct((_NP, _D), jnp.float32),
          jax.ShapeDtypeStruct((_NP, _L), jnp.float32),
      ],
  )(parts, deg, x, w1l, w1r, b1)


def _tc2_body(parts, recip, h1, w2l, w2r, b2, wpl, wpr, gidx,
              plo, pro, gpo):
  i = pl.program_id(0)
  p = parts[...]
  mean = (p[0] + p[1]) * recip[:, 0:1]
  h2 = (jnp.dot(mean, w2l[...], preferred_element_type=jnp.float32)
        + jnp.dot(h1[...], w2r[...], preferred_element_type=jnp.float32)
        + b2[...])
  plv = jnp.dot(h2, wpl[...], preferred_element_type=jnp.float32)
  prv = jnp.dot(h2, wpr[...], preferred_element_type=jnp.float32)
  plo[...] = jnp.broadcast_to(plv, (_BN, _D))
  pro[...] = jnp.broadcast_to(prv, (_BN, _L))
  oh = (gidx[...] == lax.broadcasted_iota(jnp.int32, (_BN, _G), 1))
  part_g = lax.dot_general(oh.astype(jnp.float32), h2,
                           (((0,), (0,)), ((), ())),
                           preferred_element_type=jnp.float32)

  @pl.when(i == 0)
  def _():
    gpo[...] = part_g

  @pl.when(i != 0)
  def _():
    gpo[...] += part_g


def _tc2(parts, recip, h1, w2l, w2r, b2, wpl, wpr, gidx):
  return pl.pallas_call(
      _tc2_body,
      grid=(_NB,),
      in_specs=[
          pl.BlockSpec((_NC, _BN, _D), lambda i: (0, i, 0)),
          pl.BlockSpec((_BN, _L), lambda i: (i, 0)),
          pl.BlockSpec((_BN, _D), lambda i: (i, 0)),
          pl.BlockSpec((_D, _D), lambda i: (0, 0)),
          pl.BlockSpec((_D, _D), lambda i: (0, 0)),
          pl.BlockSpec((1, _D), lambda i: (0, 0)),
          pl.BlockSpec((_D, 1), lambda i: (0, 0)),
          pl.BlockSpec((_D, 1), lambda i: (0, 0)),
          pl.BlockSpec((_BN, 1), lambda i: (i, 0)),
      ],
      out_specs=[
          pl.BlockSpec((_BN, _D), lambda i: (i, 0)),
          pl.BlockSpec((_BN, _L), lambda i: (i, 0)),
          pl.BlockSpec((_G, _D), lambda i: (0, 0)),
      ],
      out_shape=[
          jax.ShapeDtypeStruct((_NP, _D), jnp.float32),
          jax.ShapeDtypeStruct((_NP, _L), jnp.float32),
          jax.ShapeDtypeStruct((_G, _D), jnp.float32),
      ],
  )(parts, recip, h1, w2l, w2r, b2, wpl, wpr, gidx)


def _tc3_body(pparts, recip, pr, bp, gp, wv, bv, polo, valo):
  pp = pparts[...]
  ap = pp[0][:, 0:1] + pp[1][:, 0:1]
  polo[...] = ap * recip[:, 0:1] + pr[:, 0:1] + bp[0, 0]
  v = jnp.dot(gp[...], wv[...], preferred_element_type=jnp.float32) + bv[0, 0]
  valo[...] = jax.nn.sigmoid(v)


def _tc3(pparts, recip, pr, bp, gp, wv, bv):
  return pl.pallas_call(
      _tc3_body,
      out_shape=[
          jax.ShapeDtypeStruct((_NP, 1), jnp.float32),
          jax.ShapeDtypeStruct((_G, 1), jnp.float32),
      ],
  )(pparts, recip, pr, bp, gp, wv, bv)


def kernel(x, edge_index, graph_indices, W1l, W1r, b1, W2l, W2r, b2,
           Wpl, Wpr, bp, Wv, bv):
  src = edge_index[0]
  dst = edge_index[1]
  xp = jnp.pad(x, ((0, _NP - _N), (0, 0)))
  srcp = jnp.pad(src, (0, _EP - _E), constant_values=_NP - 1)
  dstp = jnp.pad(dst, (0, _EP - _E), constant_values=_NP - 1)
  gidx = jnp.pad(graph_indices, (0, _NP - _N),
                 constant_values=_G).reshape(_NP, 1)
  b1r = b1.reshape(1, _D)
  b2r = b2.reshape(1, _D)
  bpr = bp.reshape(1, 1)
  bvr = bv.reshape(1, 1)

  parts1, deg = _sc_agg_deg(xp, srcp, dstp)
  degr = deg.reshape(_NC * _NS, 1, _NP)
  h1, recip = _tc1(parts1, degr, xp, W1l, W1r, b1r)
  (parts2,) = _sc_agg(h1, srcp, dstp)
  pl2d, pr2d, gp = _tc2(parts2, recip, h1, W2l, W2r, b2r, Wpl, Wpr, gidx)
  (pparts,) = _sc_agg_pol(pl2d, srcp, dstp)
  policy, value = _tc3(pparts, recip, pr2d, bpr, gp, Wv, bvr)
  return (policy[:_N], value)


# speedup vs baseline: 1.0153x; 1.0153x over previous
import functools

import jax
import jax.numpy as jnp
from jax import lax
from jax.experimental import pallas as pl
from jax.experimental.pallas import tpu as pltpu
from jax.experimental.pallas import tpu_sc as plsc

_N = 10000
_E = 320000
_D = 128
_G = 16

_NP = 10240
_EP = 327680
_NC = 2
_NS = 16
_L = 16
_EPT = _EP // (_NC * _NS)
_CH = 128
_NG = _EPT // _CH
_RPT = _NP // _NS

_BN = 512
_NB = _NP // _BN


def _make_sc_agg(d_row, with_deg, ng0=_NG, ng1=_NG):
  assert ng0 % 4 == 0 and ng1 % 4 == 0 and 16 * (ng0 + ng1) * _CH == _EP
  out_type = [jax.ShapeDtypeStruct((_NC, _NP, d_row), jnp.float32)]
  scratch = [pltpu.VMEM_SHARED((_NP, d_row), jnp.float32)]
  if with_deg:
    out_type.append(jax.ShapeDtypeStruct((_NC, _NS, _NP), jnp.float32))
    scratch.append(pltpu.VMEM((_NP,), jnp.float32))
  scratch += [
      [pltpu.VMEM((_CH,), jnp.int32) for _ in range(4)],
      [pltpu.VMEM((_CH,), jnp.int32) for _ in range(4)],
      [pltpu.VMEM((_CH, d_row), jnp.float32) for _ in range(2)],
      [pltpu.SemaphoreType.DMA for _ in range(4)],
      [pltpu.SemaphoreType.DMA for _ in range(2)],
      [pltpu.SemaphoreType.DMA for _ in range(2)],
  ]

  def body(table_hbm, src_hbm, dst_hbm, *rest):
    if with_deg:
      (parts_out, deg_out, acc, deg_loc, sidx, didx, rows, sem_ix, sem_g,
       sem_s) = rest
    else:
      parts_out, acc, sidx, didx, rows, sem_ix, sem_g, sem_s = rest
      deg_out = deg_loc = None
    c = lax.axis_index("c")
    s = lax.axis_index("s")
    z16 = jnp.zeros((_L,), jnp.float32)
    npr = d_row // _L

    def zfill(i, carry):
      rows[0][i // npr, pl.ds((i % npr) * _L, _L)] = z16
      return carry

    lax.fori_loop(0, _CH * npr, zfill, 0)
    if with_deg:

      def dzfill(i, carry):
        deg_loc[pl.ds(i * _L, _L)] = z16
        return carry

      lax.fori_loop(0, _NP // _L, dzfill, 0)

    base = s * _RPT
    for j in range(_RPT // 128):
      pltpu.sync_copy(rows[0], acc.at[pl.ds(base + j * 128, 128), :])
    plsc.subcore_barrier()


    def issue_idx(ebase, g, slot):
      off = ebase + g * _CH
      pltpu.async_copy(src_hbm.at[pl.ds(off, _CH)], sidx[slot], sem_ix[slot])
      pltpu.async_copy(dst_hbm.at[pl.ds(off, _CH)], didx[slot], sem_ix[slot])

    def wait_idx(slot):
      pltpu.make_async_copy(src_hbm.at[pl.ds(0, _CH)], sidx[slot],
                            sem_ix[slot]).wait()
      pltpu.make_async_copy(dst_hbm.at[pl.ds(0, _CH)], didx[slot],
                            sem_ix[slot]).wait()

    def issue_gather(slot4, slot2):
      pltpu.async_copy(table_hbm.at[sidx[slot4]], rows[slot2], sem_g[slot2])

    def wait_gather(slot4, slot2):
      pltpu.make_async_copy(table_hbm.at[sidx[slot4]], rows[slot2],
                            sem_g[slot2]).wait()

    def issue_scatter(slot4, slot2):
      pltpu.async_copy(rows[slot2], acc.at[didx[slot4]], sem_s[slot2],
                       add=True)

    def wait_scatter(slot4, slot2):
      pltpu.make_async_copy(rows[slot2], acc.at[didx[slot4]],
                            sem_s[slot2]).wait()

    def run_edges(ebase, ng):
      issue_idx(ebase, 0, 0)
      issue_idx(ebase, 1, 1)
      issue_idx(ebase, 2, 2)
      wait_idx(0)
      issue_gather(0, 0)

      def pipe(i, carry):
        for b in range(4):
          b2 = b % 2
          nb2 = (b + 1) % 2
          nb4 = (b + 1) % 4
          wait_gather(b, b2)
          if with_deg:
            for k in range(_CH // _L):
              idx16 = didx[b][pl.ds(k * _L, _L)]
              cnt, lastm = plsc.scan_count(idx16)
              plsc.addupdate_scatter(deg_loc, [idx16],
                                     cnt.astype(jnp.float32), mask=lastm)
          issue_scatter(b, b2)
          if b == 0:
            @pl.when(i >= 1)
            def _():
              wait_scatter(3, nb2)
          else:
            wait_scatter(b - 1, nb2)
          if b < 3:
            wait_idx(nb4)
            issue_gather(nb4, nb2)
            lim = (ng - b) // 4
            @pl.when(i < lim)
            def _():
              issue_idx(ebase, 4 * i + b + 3, (b + 3) % 4)
          else:
            @pl.when(i < ng // 4 - 1)
            def _():
              wait_idx(0)
              issue_gather(0, nb2)
              issue_idx(ebase, 4 * i + 6, 2)
        return carry

      lax.fori_loop(0, ng // 4, pipe, 0)
      wait_scatter(3, 1)

    if ng0 == ng1:
      run_edges((c * _NS + s) * (ng0 * _CH), ng0)
    else:
      @pl.when(c == 0)
      def _():
        run_edges(s * (ng0 * _CH), ng0)

      @pl.when(c == 1)
      def _():
        run_edges(_NS * ng0 * _CH + s * (ng1 * _CH), ng1)
    if with_deg:
      pltpu.sync_copy(deg_loc, deg_out.at[c, s])
    plsc.subcore_barrier()

    for j in range(_RPT // 128):
      r0 = base + j * 128
      pltpu.sync_copy(acc.at[pl.ds(r0, 128), :],
                      parts_out.at[c, pl.ds(r0, 128), :])

  mesh = plsc.VectorSubcoreMesh(core_axis_name="c", subcore_axis_name="s",
                                num_cores=_NC, num_subcores=_NS)
  return pl.kernel(body, out_type=out_type, mesh=mesh,
                   scratch_types=scratch,
                   compiler_params=pltpu.CompilerParams(
                       needs_layout_passes=False))


_NG0 = 148
_NG1 = 12

_sc_agg_deg = _make_sc_agg(_D, True, _NG0, _NG1)
_sc_agg = _make_sc_agg(_D, False, _NG0, _NG1)
_sc_agg_pol = _make_sc_agg(_D, False, _NG0, _NG1)


def _tc1_body(parts, deg, x, w1l, w1r, b1, h1o, recipo):
  p = parts[...]
  d = jnp.sum(deg[...], axis=0)
  r_row = 1.0 / jnp.maximum(d, 1.0)
  ii = lax.broadcasted_iota(jnp.int32, (_BN, _BN), 0)
  jj = lax.broadcasted_iota(jnp.int32, (_BN, _BN), 1)
  diag_r = jnp.where(ii == jj, jnp.broadcast_to(r_row, (_BN, _BN)), 0.0)
  mean = jnp.dot(diag_r, p[0] + p[1], preferred_element_type=jnp.float32)
  r_col = jnp.sum(diag_r, axis=1, keepdims=True)
  h = (jnp.dot(mean, w1l[...], preferred_element_type=jnp.float32)
       + jnp.dot(x[...], w1r[...], preferred_element_type=jnp.float32)
       + b1[...])
  h1o[...] = jnp.maximum(h, 0.0)
  recipo[...] = jnp.broadcast_to(r_col, (_BN, _L))


def _tc1(parts, deg, x, w1l, w1r, b1):
  return pl.pallas_call(
      _tc1_body,
      grid=(_NB,),
      in_specs=[
          pl.BlockSpec((_NC, _BN, _D), lambda i: (0, i, 0)),
          pl.BlockSpec((_NC * _NS, 1, _BN), lambda i: (0, 0, i)),
          pl.BlockSpec((_BN, _D), lambda i: (i, 0)),
          pl.BlockSpec((_D, _D), lambda i: (0, 0)),
          pl.BlockSpec((_D, _D), lambda i: (0, 0)),
          pl.BlockSpec((1, _D), lambda i: (0, 0)),
      ],
      out_specs=[
          pl.BlockSpec((_BN, _D), lambda i: (i, 0)),
          pl.BlockSpec((_BN, _L), lambda i: (i, 0)),
      ],
      out_shape=[
          jax.ShapeDtypeStruct((_NP, _D), jnp.float32),
          jax.ShapeDtypeStruct((_NP, _L), jnp.float32),
      ],
  )(parts, deg, x, w1l, w1r, b1)


def _tc2_body(parts, recip, h1, w2l, w2r, b2, wpl, wpr, gidx,
              plo, pro, gpo):
  i = pl.program_id(0)
  p = parts[...]
  mean = (p[0] + p[1]) * recip[:, 0:1]
  h2 = (jnp.dot(mean, w2l[...], preferred_element_type=jnp.float32)
        + jnp.dot(h1[...], w2r[...], preferred_element_type=jnp.float32)
        + b2[...])
  plv = jnp.dot(h2, wpl[...], preferred_element_type=jnp.float32)
  prv = jnp.dot(h2, wpr[...], preferred_element_type=jnp.float32)
  plo[...] = jnp.broadcast_to(plv, (_BN, _D))
  pro[...] = jnp.broadcast_to(prv, (_BN, _L))
  oh = (gidx[...] == lax.broadcasted_iota(jnp.int32, (_BN, _G), 1))
  part_g = lax.dot_general(oh.astype(jnp.float32), h2,
                           (((0,), (0,)), ((), ())),
                           preferred_element_type=jnp.float32)

  @pl.when(i == 0)
  def _():
    gpo[...] = part_g

  @pl.when(i != 0)
  def _():
    gpo[...] += part_g


def _tc2(parts, recip, h1, w2l, w2r, b2, wpl, wpr, gidx):
  return pl.pallas_call(
      _tc2_body,
      grid=(_NB,),
      in_specs=[
          pl.BlockSpec((_NC, _BN, _D), lambda i: (0, i, 0)),
          pl.BlockSpec((_BN, _L), lambda i: (i, 0)),
          pl.BlockSpec((_BN, _D), lambda i: (i, 0)),
          pl.BlockSpec((_D, _D), lambda i: (0, 0)),
          pl.BlockSpec((_D, _D), lambda i: (0, 0)),
          pl.BlockSpec((1, _D), lambda i: (0, 0)),
          pl.BlockSpec((_D, 1), lambda i: (0, 0)),
          pl.BlockSpec((_D, 1), lambda i: (0, 0)),
          pl.BlockSpec((_BN, 1), lambda i: (i, 0)),
      ],
      out_specs=[
          pl.BlockSpec((_BN, _D), lambda i: (i, 0)),
          pl.BlockSpec((_BN, _L), lambda i: (i, 0)),
          pl.BlockSpec((_G, _D), lambda i: (0, 0)),
      ],
      out_shape=[
          jax.ShapeDtypeStruct((_NP, _D), jnp.float32),
          jax.ShapeDtypeStruct((_NP, _L), jnp.float32),
          jax.ShapeDtypeStruct((_G, _D), jnp.float32),
      ],
  )(parts, recip, h1, w2l, w2r, b2, wpl, wpr, gidx)


def _tc3_body(pparts, recip, pr, bp, gp, wv, bv, polo, valo):
  pp = pparts[...]
  ap = pp[0][:, 0:1] + pp[1][:, 0:1]
  polo[...] = ap * recip[:, 0:1] + pr[:, 0:1] + bp[0, 0]
  v = jnp.dot(gp[...], wv[...], preferred_element_type=jnp.float32) + bv[0, 0]
  valo[...] = jax.nn.sigmoid(v)


def _tc3(pparts, recip, pr, bp, gp, wv, bv):
  return pl.pallas_call(
      _tc3_body,
      out_shape=[
          jax.ShapeDtypeStruct((_NP, 1), jnp.float32),
          jax.ShapeDtypeStruct((_G, 1), jnp.float32),
      ],
  )(pparts, recip, pr, bp, gp, wv, bv)


def kernel(x, edge_index, graph_indices, W1l, W1r, b1, W2l, W2r, b2,
           Wpl, Wpr, bp, Wv, bv):
  src = edge_index[0]
  dst = edge_index[1]
  xp = jnp.pad(x, ((0, _NP - _N), (0, 0)))
  srcp = jnp.pad(src, (0, _EP - _E), constant_values=_NP - 1)
  dstp = jnp.pad(dst, (0, _EP - _E), constant_values=_NP - 1)
  gidx = jnp.pad(graph_indices, (0, _NP - _N),
                 constant_values=_G).reshape(_NP, 1)
  b1r = b1.reshape(1, _D)
  b2r = b2.reshape(1, _D)
  bpr = bp.reshape(1, 1)
  bvr = bv.reshape(1, 1)

  parts1, deg = _sc_agg_deg(xp, srcp, dstp)
  degr = deg.reshape(_NC * _NS, 1, _NP)
  h1, recip = _tc1(parts1, degr, xp, W1l, W1r, b1r)
  (parts2,) = _sc_agg(h1, srcp, dstp)
  pl2d, pr2d, gp = _tc2(parts2, recip, h1, W2l, W2r, b2r, Wpl, Wpr, gidx)
  (pparts,) = _sc_agg_pol(pl2d, srcp, dstp)
  policy, value = _tc3(pparts, recip, pr2d, bpr, gp, Wv, bvr)
  return (policy[:_N], value)


# speedup vs baseline: 1.3993x; 1.3782x over previous
import functools

import jax
import jax.numpy as jnp
from jax import lax
from jax.experimental import pallas as pl
from jax.experimental.pallas import tpu as pltpu
from jax.experimental.pallas import tpu_sc as plsc

_N = 10000
_E = 320000
_D = 128
_G = 16

_NP = 10240
_EP = 327680
_NC = 2
_NS = 16
_L = 16
_EPT = _EP // (_NC * _NS)
_CH = 128
_NG = _EPT // _CH
_RPT = _NP // _NS

_BN = 512
_NB = _NP // _BN


def _make_sc_agg(d_row, with_deg, ng0=_NG, ng1=_NG):
  assert ng0 % 4 == 0 and ng1 % 4 == 0 and 16 * (ng0 + ng1) * _CH == _EP
  out_type = [jax.ShapeDtypeStruct((_NC, _NP, d_row), jnp.float32)]
  scratch = [pltpu.VMEM_SHARED((_NP, d_row), jnp.float32)]
  if with_deg:
    out_type.append(jax.ShapeDtypeStruct((_NC, _NS, _NP), jnp.float32))
    scratch.append(pltpu.VMEM((_NP,), jnp.float32))
  scratch += [
      [pltpu.VMEM((_CH,), jnp.int32) for _ in range(4)],
      [pltpu.VMEM((_CH,), jnp.int32) for _ in range(4)],
      [pltpu.VMEM((_CH, d_row), jnp.float32) for _ in range(2)],
      [pltpu.SemaphoreType.DMA for _ in range(4)],
      [pltpu.SemaphoreType.DMA for _ in range(2)],
      [pltpu.SemaphoreType.DMA for _ in range(2)],
  ]

  def body(table_hbm, src_hbm, dst_hbm, *rest):
    if with_deg:
      (parts_out, deg_out, acc, deg_loc, sidx, didx, rows, sem_ix, sem_g,
       sem_s) = rest
    else:
      parts_out, acc, sidx, didx, rows, sem_ix, sem_g, sem_s = rest
      deg_out = deg_loc = None
    c = lax.axis_index("c")
    s = lax.axis_index("s")
    z16 = jnp.zeros((_L,), jnp.float32)
    npr = d_row // _L

    def zfill(i, carry):
      rows[0][i // npr, pl.ds((i % npr) * _L, _L)] = z16
      return carry

    lax.fori_loop(0, _CH * npr, zfill, 0)
    if with_deg:

      def dzfill(i, carry):
        deg_loc[pl.ds(i * _L, _L)] = z16
        return carry

      lax.fori_loop(0, _NP // _L, dzfill, 0)

    base = s * _RPT
    for j in range(_RPT // 128):
      pltpu.sync_copy(rows[0], acc.at[pl.ds(base + j * 128, 128), :])
    plsc.subcore_barrier()


    def issue_idx(ebase, g, slot):
      off = ebase + g * _CH
      pltpu.async_copy(src_hbm.at[pl.ds(off, _CH)], sidx[slot], sem_ix[slot])
      pltpu.async_copy(dst_hbm.at[pl.ds(off, _CH)], didx[slot], sem_ix[slot])

    def wait_idx(slot):
      pltpu.make_async_copy(src_hbm.at[pl.ds(0, _CH)], sidx[slot],
                            sem_ix[slot]).wait()
      pltpu.make_async_copy(dst_hbm.at[pl.ds(0, _CH)], didx[slot],
                            sem_ix[slot]).wait()

    def issue_gather(slot4, slot2):
      pltpu.async_copy(table_hbm.at[sidx[slot4]], rows[slot2], sem_g[slot2])

    def wait_gather(slot4, slot2):
      pltpu.make_async_copy(table_hbm.at[sidx[slot4]], rows[slot2],
                            sem_g[slot2]).wait()

    def issue_scatter(slot4, slot2):
      pltpu.async_copy(rows[slot2], acc.at[didx[slot4]], sem_s[slot2],
                       add=True)

    def wait_scatter(slot4, slot2):
      pltpu.make_async_copy(rows[slot2], acc.at[didx[slot4]],
                            sem_s[slot2]).wait()

    def run_edges(ebase, ng):
      issue_idx(ebase, 0, 0)
      issue_idx(ebase, 1, 1)
      issue_idx(ebase, 2, 2)
      wait_idx(0)
      issue_gather(0, 0)

      def pipe(i, carry):
        for b in range(4):
          b2 = b % 2
          nb2 = (b + 1) % 2
          nb4 = (b + 1) % 4
          wait_gather(b, b2)
          if with_deg:
            for k in range(_CH // _L):
              idx16 = didx[b][pl.ds(k * _L, _L)]
              cnt, lastm = plsc.scan_count(idx16)
              plsc.addupdate_scatter(deg_loc, [idx16],
                                     cnt.astype(jnp.float32), mask=lastm)
          issue_scatter(b, b2)
          if b == 0:
            @pl.when(i >= 1)
            def _():
              wait_scatter(3, nb2)
          else:
            wait_scatter(b - 1, nb2)
          if b < 3:
            wait_idx(nb4)
            issue_gather(nb4, nb2)
            lim = (ng - b) // 4
            @pl.when(i < lim)
            def _():
              issue_idx(ebase, 4 * i + b + 3, (b + 3) % 4)
          else:
            @pl.when(i < ng // 4 - 1)
            def _():
              wait_idx(0)
              issue_gather(0, nb2)
              issue_idx(ebase, 4 * i + 6, 2)
        return carry

      lax.fori_loop(0, ng // 4, pipe, 0)
      wait_scatter(3, 1)

    if ng0 == ng1:
      run_edges((c * _NS + s) * (ng0 * _CH), ng0)
    else:
      @pl.when(c == 0)
      def _():
        run_edges(s * (ng0 * _CH), ng0)

      @pl.when(c == 1)
      def _():
        run_edges(_NS * ng0 * _CH + s * (ng1 * _CH), ng1)
    if with_deg:
      pltpu.sync_copy(deg_loc, deg_out.at[c, s])
    plsc.subcore_barrier()

    for j in range(_RPT // 128):
      r0 = base + j * 128
      pltpu.sync_copy(acc.at[pl.ds(r0, 128), :],
                      parts_out.at[c, pl.ds(r0, 128), :])

  mesh = plsc.VectorSubcoreMesh(core_axis_name="c", subcore_axis_name="s",
                                num_cores=_NC, num_subcores=_NS)
  return pl.kernel(body, out_type=out_type, mesh=mesh,
                   scratch_types=scratch,
                   compiler_params=pltpu.CompilerParams(
                       needs_layout_passes=False))


_NG0 = 148
_NG1 = 12

_sc_agg_deg = _make_sc_agg(_D, True, _NG0, _NG1)
_sc_agg = _make_sc_agg(_D, False, _NG0, _NG1)


def _sc_pol_body(plv_hbm, src_hbm, dst_hbm, out, plv_loc, agg_loc,
                 sidx_all, didx_all, sem):
  c = lax.axis_index("c")
  s = lax.axis_index("s")
  z16 = jnp.zeros((_L,), jnp.float32)

  def zf(i, carry):
    agg_loc[pl.ds(i * _L, _L)] = z16
    return carry

  lax.fori_loop(0, _NP // _L, zf, 0)
  ebase = (c * _NS + s) * _EPT
  pltpu.async_copy(src_hbm.at[pl.ds(ebase, _EPT)], sidx_all, sem)
  pltpu.async_copy(dst_hbm.at[pl.ds(ebase, _EPT)], didx_all, sem)
  pltpu.async_copy(plv_hbm, plv_loc, sem)
  pltpu.make_async_copy(src_hbm.at[pl.ds(0, _EPT)], sidx_all, sem).wait()
  pltpu.make_async_copy(dst_hbm.at[pl.ds(0, _EPT)], didx_all, sem).wait()
  pltpu.make_async_copy(plv_hbm, plv_loc, sem).wait()
  iota = lax.broadcasted_iota(jnp.int32, (_L,), 0)
  masks = [iota == k for k in range(_L)]

  def step(i, carry):
    s16 = sidx_all[pl.ds(i * _L, _L)]
    d16 = didx_all[pl.ds(i * _L, _L)]
    vals = plsc.load_gather(plv_loc, [s16])
    for k in range(_L):
      plsc.addupdate_scatter(agg_loc, [d16], vals, mask=masks[k])
    return carry

  lax.fori_loop(0, _EPT // _L, step, 0)
  pltpu.sync_copy(agg_loc, out.at[c, s])


_sc_pol_local = pl.kernel(
    _sc_pol_body,
    out_type=[jax.ShapeDtypeStruct((_NC, _NS, _NP), jnp.float32)],
    mesh=plsc.VectorSubcoreMesh(core_axis_name="c", subcore_axis_name="s",
                                num_cores=_NC, num_subcores=_NS),
    scratch_types=[
        pltpu.VMEM((_NP,), jnp.float32),
        pltpu.VMEM((_NP,), jnp.float32),
        pltpu.VMEM((_EPT,), jnp.int32),
        pltpu.VMEM((_EPT,), jnp.int32),
        pltpu.SemaphoreType.DMA,
    ],
    compiler_params=pltpu.CompilerParams(needs_layout_passes=False),
)


def _tc1_body(parts, deg, x, w1l, w1r, b1, h1o, recipo):
  p = parts[...]
  d = jnp.sum(deg[...], axis=0)
  r_row = 1.0 / jnp.maximum(d, 1.0)
  ii = lax.broadcasted_iota(jnp.int32, (_BN, _BN), 0)
  jj = lax.broadcasted_iota(jnp.int32, (_BN, _BN), 1)
  diag_r = jnp.where(ii == jj, jnp.broadcast_to(r_row, (_BN, _BN)), 0.0)
  mean = jnp.dot(diag_r, p[0] + p[1], preferred_element_type=jnp.float32)
  r_col = jnp.sum(diag_r, axis=1, keepdims=True)
  h = (jnp.dot(mean, w1l[...], preferred_element_type=jnp.float32)
       + jnp.dot(x[...], w1r[...], preferred_element_type=jnp.float32)
       + b1[...])
  h1o[...] = jnp.maximum(h, 0.0)
  recipo[...] = jnp.broadcast_to(r_col, (_BN, _L))


def _tc1(parts, deg, x, w1l, w1r, b1):
  return pl.pallas_call(
      _tc1_body,
      grid=(_NB,),
      in_specs=[
          pl.BlockSpec((_NC, _BN, _D), lambda i: (0, i, 0)),
          pl.BlockSpec((_NC * _NS, 1, _BN), lambda i: (0, 0, i)),
          pl.BlockSpec((_BN, _D), lambda i: (i, 0)),
          pl.BlockSpec((_D, _D), lambda i: (0, 0)),
          pl.BlockSpec((_D, _D), lambda i: (0, 0)),
          pl.BlockSpec((1, _D), lambda i: (0, 0)),
      ],
      out_specs=[
          pl.BlockSpec((_BN, _D), lambda i: (i, 0)),
          pl.BlockSpec((_BN, _L), lambda i: (i, 0)),
      ],
      out_shape=[
          jax.ShapeDtypeStruct((_NP, _D), jnp.float32),
          jax.ShapeDtypeStruct((_NP, _L), jnp.float32),
      ],
  )(parts, deg, x, w1l, w1r, b1)


def _tc2_body(parts, recip, h1, w2l, w2r, b2, wpl, wpr, gidx,
              plo, pro, gpo):
  i = pl.program_id(0)
  p = parts[...]
  mean = (p[0] + p[1]) * recip[:, 0:1]
  h2 = (jnp.dot(mean, w2l[...], preferred_element_type=jnp.float32)
        + jnp.dot(h1[...], w2r[...], preferred_element_type=jnp.float32)
        + b2[...])
  plv = jnp.dot(h2, wpl[...], preferred_element_type=jnp.float32)
  prv = jnp.dot(h2, wpr[...], preferred_element_type=jnp.float32)
  di = lax.broadcasted_iota(jnp.int32, (_BN, _BN), 0)
  dj = lax.broadcasted_iota(jnp.int32, (_BN, _BN), 1)
  diag_p = jnp.where(di == dj, jnp.broadcast_to(plv, (_BN, _BN)), 0.0)
  plo[...] = jnp.sum(diag_p, axis=0, keepdims=True)[None]
  pro[...] = jnp.broadcast_to(prv, (_BN, _L))
  oh = (gidx[...] == lax.broadcasted_iota(jnp.int32, (_BN, _G), 1))
  part_g = lax.dot_general(oh.astype(jnp.float32), h2,
                           (((0,), (0,)), ((), ())),
                           preferred_element_type=jnp.float32)

  @pl.when(i == 0)
  def _():
    gpo[...] = part_g

  @pl.when(i != 0)
  def _():
    gpo[...] += part_g


def _tc2(parts, recip, h1, w2l, w2r, b2, wpl, wpr, gidx):
  return pl.pallas_call(
      _tc2_body,
      grid=(_NB,),
      in_specs=[
          pl.BlockSpec((_NC, _BN, _D), lambda i: (0, i, 0)),
          pl.BlockSpec((_BN, _L), lambda i: (i, 0)),
          pl.BlockSpec((_BN, _D), lambda i: (i, 0)),
          pl.BlockSpec((_D, _D), lambda i: (0, 0)),
          pl.BlockSpec((_D, _D), lambda i: (0, 0)),
          pl.BlockSpec((1, _D), lambda i: (0, 0)),
          pl.BlockSpec((_D, 1), lambda i: (0, 0)),
          pl.BlockSpec((_D, 1), lambda i: (0, 0)),
          pl.BlockSpec((_BN, 1), lambda i: (i, 0)),
      ],
      out_specs=[
          pl.BlockSpec((1, 1, _BN), lambda i: (i, 0, 0)),
          pl.BlockSpec((_BN, _L), lambda i: (i, 0)),
          pl.BlockSpec((_G, _D), lambda i: (0, 0)),
      ],
      out_shape=[
          jax.ShapeDtypeStruct((_NB, 1, _BN), jnp.float32),
          jax.ShapeDtypeStruct((_NP, _L), jnp.float32),
          jax.ShapeDtypeStruct((_G, _D), jnp.float32),
      ],
  )(parts, recip, h1, w2l, w2r, b2, wpl, wpr, gidx)


def _tc3_body(pp4, recip, pr, bp, gp, wv, bv, polo, valo):
  i = pl.program_id(0)
  ap_row = jnp.sum(pp4[...], axis=(0, 1))
  ii = lax.broadcasted_iota(jnp.int32, (_BN, _BN), 0)
  jj = lax.broadcasted_iota(jnp.int32, (_BN, _BN), 1)
  diag_a = jnp.where(ii == jj, jnp.broadcast_to(ap_row, (_BN, _BN)), 0.0)
  ap_col = jnp.sum(diag_a, axis=1, keepdims=True)
  polo[...] = ap_col * recip[:, 0:1] + pr[:, 0:1] + bp[0, 0]

  @pl.when(i == 0)
  def _():
    v = (jnp.dot(gp[...], wv[...], preferred_element_type=jnp.float32)
         + bv[0, 0])
    valo[...] = jax.nn.sigmoid(v)


def _tc3(pp4, recip, pr, bp, gp, wv, bv):
  return pl.pallas_call(
      _tc3_body,
      grid=(_NB,),
      in_specs=[
          pl.BlockSpec((_NC, _NS, 1, _BN), lambda i: (0, 0, 0, i)),
          pl.BlockSpec((_BN, _L), lambda i: (i, 0)),
          pl.BlockSpec((_BN, _L), lambda i: (i, 0)),
          pl.BlockSpec((1, 1), lambda i: (0, 0)),
          pl.BlockSpec((_G, _D), lambda i: (0, 0)),
          pl.BlockSpec((_D, 1), lambda i: (0, 0)),
          pl.BlockSpec((1, 1), lambda i: (0, 0)),
      ],
      out_specs=[
          pl.BlockSpec((_BN, 1), lambda i: (i, 0)),
          pl.BlockSpec((_G, 1), lambda i: (0, 0)),
      ],
      out_shape=[
          jax.ShapeDtypeStruct((_NP, 1), jnp.float32),
          jax.ShapeDtypeStruct((_G, 1), jnp.float32),
      ],
  )(pp4, recip, pr, bp, gp, wv, bv)


def kernel(x, edge_index, graph_indices, W1l, W1r, b1, W2l, W2r, b2,
           Wpl, Wpr, bp, Wv, bv):
  src = edge_index[0]
  dst = edge_index[1]
  xp = jnp.pad(x, ((0, _NP - _N), (0, 0)))
  srcp = jnp.pad(src, (0, _EP - _E), constant_values=_NP - 1)
  dstp = jnp.pad(dst, (0, _EP - _E), constant_values=_NP - 1)
  gidx = jnp.pad(graph_indices, (0, _NP - _N),
                 constant_values=_G).reshape(_NP, 1)
  b1r = b1.reshape(1, _D)
  b2r = b2.reshape(1, _D)
  bpr = bp.reshape(1, 1)
  bvr = bv.reshape(1, 1)

  parts1, deg = _sc_agg_deg(xp, srcp, dstp)
  degr = deg.reshape(_NC * _NS, 1, _NP)
  h1, recip = _tc1(parts1, degr, xp, W1l, W1r, b1r)
  (parts2,) = _sc_agg(h1, srcp, dstp)
  pl_lane, pr2d, gp = _tc2(parts2, recip, h1, W2l, W2r, b2r, Wpl, Wpr, gidx)
  (pparts3,) = _sc_pol_local(pl_lane.reshape(_NP), srcp, dstp)
  pp4 = pparts3.reshape(_NC, _NS, 1, _NP)
  policy, value = _tc3(pp4, recip, pr2d, bpr, gp, Wv, bvr)
  return (policy[:_N], value)


# speedup vs baseline: 1.4042x; 1.0035x over previous
import functools

import jax
import jax.numpy as jnp
from jax import lax
from jax.experimental import pallas as pl
from jax.experimental.pallas import tpu as pltpu
from jax.experimental.pallas import tpu_sc as plsc

_N = 10000
_E = 320000
_D = 128
_G = 16

_NP = 10240
_EP = 327680
_NC = 2
_NS = 16
_L = 16
_EPT = _EP // (_NC * _NS)
_CH = 128
_NG = _EPT // _CH
_RPT = _NP // _NS

_BN = 512
_NB = _NP // _BN


def _make_sc_agg(d_row, with_deg, ng0=_NG, ng1=_NG):
  assert ng0 % 4 == 0 and ng1 % 4 == 0 and 16 * (ng0 + ng1) * _CH == _EP
  out_type = [jax.ShapeDtypeStruct((_NC, _NP, d_row), jnp.float32)]
  scratch = [pltpu.VMEM_SHARED((_NP, d_row), jnp.float32)]
  if with_deg:
    out_type.append(jax.ShapeDtypeStruct((_NC, _NS, _NP), jnp.float32))
    scratch.append(pltpu.VMEM((_NP,), jnp.float32))
  scratch += [
      [pltpu.VMEM((_CH,), jnp.int32) for _ in range(4)],
      [pltpu.VMEM((_CH,), jnp.int32) for _ in range(4)],
      [pltpu.VMEM((_CH, d_row), jnp.float32) for _ in range(2)],
      [pltpu.SemaphoreType.DMA for _ in range(4)],
      [pltpu.SemaphoreType.DMA for _ in range(2)],
      [pltpu.SemaphoreType.DMA for _ in range(2)],
  ]

  def body(table_hbm, src_hbm, dst_hbm, *rest):
    if with_deg:
      (parts_out, deg_out, acc, deg_loc, sidx, didx, rows, sem_ix, sem_g,
       sem_s) = rest
    else:
      parts_out, acc, sidx, didx, rows, sem_ix, sem_g, sem_s = rest
      deg_out = deg_loc = None
    c = lax.axis_index("c")
    s = lax.axis_index("s")
    z16 = jnp.zeros((_L,), jnp.float32)
    npr = d_row // _L

    def zfill(i, carry):
      rows[0][i // npr, pl.ds((i % npr) * _L, _L)] = z16
      return carry

    lax.fori_loop(0, _CH * npr, zfill, 0)
    if with_deg:

      def dzfill(i, carry):
        deg_loc[pl.ds(i * _L, _L)] = z16
        return carry

      lax.fori_loop(0, _NP // _L, dzfill, 0)

    base = s * _RPT
    for j in range(_RPT // 128):
      pltpu.sync_copy(rows[0], acc.at[pl.ds(base + j * 128, 128), :])
    plsc.subcore_barrier()


    def issue_idx(ebase, g, slot):
      off = ebase + g * _CH
      pltpu.async_copy(src_hbm.at[pl.ds(off, _CH)], sidx[slot], sem_ix[slot])
      pltpu.async_copy(dst_hbm.at[pl.ds(off, _CH)], didx[slot], sem_ix[slot])

    def wait_idx(slot):
      pltpu.make_async_copy(src_hbm.at[pl.ds(0, _CH)], sidx[slot],
                            sem_ix[slot]).wait()
      pltpu.make_async_copy(dst_hbm.at[pl.ds(0, _CH)], didx[slot],
                            sem_ix[slot]).wait()

    def issue_gather(slot4, slot2):
      pltpu.async_copy(table_hbm.at[sidx[slot4]], rows[slot2], sem_g[slot2])

    def wait_gather(slot4, slot2):
      pltpu.make_async_copy(table_hbm.at[sidx[slot4]], rows[slot2],
                            sem_g[slot2]).wait()

    def issue_scatter(slot4, slot2):
      pltpu.async_copy(rows[slot2], acc.at[didx[slot4]], sem_s[slot2],
                       add=True)

    def wait_scatter(slot4, slot2):
      pltpu.make_async_copy(rows[slot2], acc.at[didx[slot4]],
                            sem_s[slot2]).wait()

    def run_edges(ebase, ng):
      issue_idx(ebase, 0, 0)
      issue_idx(ebase, 1, 1)
      issue_idx(ebase, 2, 2)
      wait_idx(0)
      issue_gather(0, 0)

      def pipe(i, carry):
        for b in range(4):
          b2 = b % 2
          nb2 = (b + 1) % 2
          nb4 = (b + 1) % 4
          if b == 0:
            @pl.when(i >= 1)
            def _():
              wait_scatter(3, nb2)
          else:
            wait_scatter(b - 1, nb2)
          if b < 3:
            wait_idx(nb4)
            issue_gather(nb4, nb2)
            lim = (ng - b) // 4
            @pl.when(i < lim)
            def _():
              issue_idx(ebase, 4 * i + b + 3, (b + 3) % 4)
          else:
            @pl.when(i < ng // 4 - 1)
            def _():
              wait_idx(0)
              issue_gather(0, nb2)
              issue_idx(ebase, 4 * i + 6, 2)
          wait_gather(b, b2)
          if with_deg:
            for k in range(_CH // _L):
              idx16 = didx[b][pl.ds(k * _L, _L)]
              cnt, lastm = plsc.scan_count(idx16)
              plsc.addupdate_scatter(deg_loc, [idx16],
                                     cnt.astype(jnp.float32), mask=lastm)
          issue_scatter(b, b2)
        return carry

      lax.fori_loop(0, ng // 4, pipe, 0)
      wait_scatter(3, 1)

    if ng0 == ng1:
      run_edges((c * _NS + s) * (ng0 * _CH), ng0)
    else:
      @pl.when(c == 0)
      def _():
        run_edges(s * (ng0 * _CH), ng0)

      @pl.when(c == 1)
      def _():
        run_edges(_NS * ng0 * _CH + s * (ng1 * _CH), ng1)
    if with_deg:
      pltpu.sync_copy(deg_loc, deg_out.at[c, s])
    plsc.subcore_barrier()

    for j in range(_RPT // 128):
      r0 = base + j * 128
      pltpu.sync_copy(acc.at[pl.ds(r0, 128), :],
                      parts_out.at[c, pl.ds(r0, 128), :])

  mesh = plsc.VectorSubcoreMesh(core_axis_name="c", subcore_axis_name="s",
                                num_cores=_NC, num_subcores=_NS)
  return pl.kernel(body, out_type=out_type, mesh=mesh,
                   scratch_types=scratch,
                   compiler_params=pltpu.CompilerParams(
                       needs_layout_passes=False))


_NG0 = 148
_NG1 = 12

_sc_agg_deg = _make_sc_agg(_D, True, _NG0, _NG1)
_sc_agg = _make_sc_agg(_D, False, _NG0, _NG1)


def _sc_pol_body(plv_hbm, src_hbm, dst_hbm, out, plv_loc, agg_loc,
                 sidx_all, didx_all, sem):
  c = lax.axis_index("c")
  s = lax.axis_index("s")
  z16 = jnp.zeros((_L,), jnp.float32)

  def zf(i, carry):
    agg_loc[pl.ds(i * _L, _L)] = z16
    return carry

  lax.fori_loop(0, _NP // _L, zf, 0)
  ebase = (c * _NS + s) * _EPT
  pltpu.async_copy(src_hbm.at[pl.ds(ebase, _EPT)], sidx_all, sem)
  pltpu.async_copy(dst_hbm.at[pl.ds(ebase, _EPT)], didx_all, sem)
  pltpu.async_copy(plv_hbm, plv_loc, sem)
  pltpu.make_async_copy(src_hbm.at[pl.ds(0, _EPT)], sidx_all, sem).wait()
  pltpu.make_async_copy(dst_hbm.at[pl.ds(0, _EPT)], didx_all, sem).wait()
  pltpu.make_async_copy(plv_hbm, plv_loc, sem).wait()
  iota = lax.broadcasted_iota(jnp.int32, (_L,), 0)
  masks = [iota == k for k in range(_L)]

  def step(i, carry):
    s16 = sidx_all[pl.ds(i * _L, _L)]
    d16 = didx_all[pl.ds(i * _L, _L)]
    vals = plsc.load_gather(plv_loc, [s16])
    for k in range(_L):
      plsc.addupdate_scatter(agg_loc, [d16], vals, mask=masks[k])
    return carry

  lax.fori_loop(0, _EPT // _L, step, 0)
  pltpu.sync_copy(agg_loc, out.at[c, s])


_sc_pol_local = pl.kernel(
    _sc_pol_body,
    out_type=[jax.ShapeDtypeStruct((_NC, _NS, _NP), jnp.float32)],
    mesh=plsc.VectorSubcoreMesh(core_axis_name="c", subcore_axis_name="s",
                                num_cores=_NC, num_subcores=_NS),
    scratch_types=[
        pltpu.VMEM((_NP,), jnp.float32),
        pltpu.VMEM((_NP,), jnp.float32),
        pltpu.VMEM((_EPT,), jnp.int32),
        pltpu.VMEM((_EPT,), jnp.int32),
        pltpu.SemaphoreType.DMA,
    ],
    compiler_params=pltpu.CompilerParams(needs_layout_passes=False),
)


def _tc1_body(parts, deg, x, w1l, w1r, b1, h1o, recipo):
  p = parts[...]
  d = jnp.sum(deg[...], axis=0)
  r_row = 1.0 / jnp.maximum(d, 1.0)
  ii = lax.broadcasted_iota(jnp.int32, (_BN, _BN), 0)
  jj = lax.broadcasted_iota(jnp.int32, (_BN, _BN), 1)
  diag_r = jnp.where(ii == jj, jnp.broadcast_to(r_row, (_BN, _BN)), 0.0)
  mean = jnp.dot(diag_r, p[0] + p[1], preferred_element_type=jnp.float32)
  r_col = jnp.sum(diag_r, axis=1, keepdims=True)
  h = (jnp.dot(mean, w1l[...], preferred_element_type=jnp.float32)
       + jnp.dot(x[...], w1r[...], preferred_element_type=jnp.float32)
       + b1[...])
  h1o[...] = jnp.maximum(h, 0.0)
  recipo[...] = jnp.broadcast_to(r_col, (_BN, _L))


def _tc1(parts, deg, x, w1l, w1r, b1):
  return pl.pallas_call(
      _tc1_body,
      grid=(_NB,),
      in_specs=[
          pl.BlockSpec((_NC, _BN, _D), lambda i: (0, i, 0)),
          pl.BlockSpec((_NC * _NS, 1, _BN), lambda i: (0, 0, i)),
          pl.BlockSpec((_BN, _D), lambda i: (i, 0)),
          pl.BlockSpec((_D, _D), lambda i: (0, 0)),
          pl.BlockSpec((_D, _D), lambda i: (0, 0)),
          pl.BlockSpec((1, _D), lambda i: (0, 0)),
      ],
      out_specs=[
          pl.BlockSpec((_BN, _D), lambda i: (i, 0)),
          pl.BlockSpec((_BN, _L), lambda i: (i, 0)),
      ],
      out_shape=[
          jax.ShapeDtypeStruct((_NP, _D), jnp.float32),
          jax.ShapeDtypeStruct((_NP, _L), jnp.float32),
      ],
  )(parts, deg, x, w1l, w1r, b1)


def _tc2_body(parts, recip, h1, w2l, w2r, b2, wpl, wpr, gidx,
              plo, pro, gpo):
  i = pl.program_id(0)
  p = parts[...]
  mean = (p[0] + p[1]) * recip[:, 0:1]
  h2 = (jnp.dot(mean, w2l[...], preferred_element_type=jnp.float32)
        + jnp.dot(h1[...], w2r[...], preferred_element_type=jnp.float32)
        + b2[...])
  plv = jnp.dot(h2, wpl[...], preferred_element_type=jnp.float32)
  prv = jnp.dot(h2, wpr[...], preferred_element_type=jnp.float32)
  di = lax.broadcasted_iota(jnp.int32, (_BN, _BN), 0)
  dj = lax.broadcasted_iota(jnp.int32, (_BN, _BN), 1)
  diag_p = jnp.where(di == dj, jnp.broadcast_to(plv, (_BN, _BN)), 0.0)
  plo[...] = jnp.sum(diag_p, axis=0, keepdims=True)[None]
  pro[...] = jnp.broadcast_to(prv, (_BN, _L))
  oh = (gidx[...] == lax.broadcasted_iota(jnp.int32, (_BN, _G), 1))
  part_g = lax.dot_general(oh.astype(jnp.float32), h2,
                           (((0,), (0,)), ((), ())),
                           preferred_element_type=jnp.float32)

  @pl.when(i == 0)
  def _():
    gpo[...] = part_g

  @pl.when(i != 0)
  def _():
    gpo[...] += part_g


def _tc2(parts, recip, h1, w2l, w2r, b2, wpl, wpr, gidx):
  return pl.pallas_call(
      _tc2_body,
      grid=(_NB,),
      in_specs=[
          pl.BlockSpec((_NC, _BN, _D), lambda i: (0, i, 0)),
          pl.BlockSpec((_BN, _L), lambda i: (i, 0)),
          pl.BlockSpec((_BN, _D), lambda i: (i, 0)),
          pl.BlockSpec((_D, _D), lambda i: (0, 0)),
          pl.BlockSpec((_D, _D), lambda i: (0, 0)),
          pl.BlockSpec((1, _D), lambda i: (0, 0)),
          pl.BlockSpec((_D, 1), lambda i: (0, 0)),
          pl.BlockSpec((_D, 1), lambda i: (0, 0)),
          pl.BlockSpec((_BN, 1), lambda i: (i, 0)),
      ],
      out_specs=[
          pl.BlockSpec((1, 1, _BN), lambda i: (i, 0, 0)),
          pl.BlockSpec((_BN, _L), lambda i: (i, 0)),
          pl.BlockSpec((_G, _D), lambda i: (0, 0)),
      ],
      out_shape=[
          jax.ShapeDtypeStruct((_NB, 1, _BN), jnp.float32),
          jax.ShapeDtypeStruct((_NP, _L), jnp.float32),
          jax.ShapeDtypeStruct((_G, _D), jnp.float32),
      ],
  )(parts, recip, h1, w2l, w2r, b2, wpl, wpr, gidx)


def _tc3_body(pp4, recip, pr, bp, gp, wv, bv, polo, valo):
  i = pl.program_id(0)
  ap_row = jnp.sum(pp4[...], axis=(0, 1))
  ii = lax.broadcasted_iota(jnp.int32, (_BN, _BN), 0)
  jj = lax.broadcasted_iota(jnp.int32, (_BN, _BN), 1)
  diag_a = jnp.where(ii == jj, jnp.broadcast_to(ap_row, (_BN, _BN)), 0.0)
  ap_col = jnp.sum(diag_a, axis=1, keepdims=True)
  polo[...] = ap_col * recip[:, 0:1] + pr[:, 0:1] + bp[0, 0]

  @pl.when(i == 0)
  def _():
    v = (jnp.dot(gp[...], wv[...], preferred_element_type=jnp.float32)
         + bv[0, 0])
    valo[...] = jax.nn.sigmoid(v)


def _tc3(pp4, recip, pr, bp, gp, wv, bv):
  return pl.pallas_call(
      _tc3_body,
      grid=(_NB,),
      in_specs=[
          pl.BlockSpec((_NC, _NS, 1, _BN), lambda i: (0, 0, 0, i)),
          pl.BlockSpec((_BN, _L), lambda i: (i, 0)),
          pl.BlockSpec((_BN, _L), lambda i: (i, 0)),
          pl.BlockSpec((1, 1), lambda i: (0, 0)),
          pl.BlockSpec((_G, _D), lambda i: (0, 0)),
          pl.BlockSpec((_D, 1), lambda i: (0, 0)),
          pl.BlockSpec((1, 1), lambda i: (0, 0)),
      ],
      out_specs=[
          pl.BlockSpec((_BN, 1), lambda i: (i, 0)),
          pl.BlockSpec((_G, 1), lambda i: (0, 0)),
      ],
      out_shape=[
          jax.ShapeDtypeStruct((_NP, 1), jnp.float32),
          jax.ShapeDtypeStruct((_G, 1), jnp.float32),
      ],
  )(pp4, recip, pr, bp, gp, wv, bv)


def kernel(x, edge_index, graph_indices, W1l, W1r, b1, W2l, W2r, b2,
           Wpl, Wpr, bp, Wv, bv):
  src = edge_index[0]
  dst = edge_index[1]
  xp = jnp.pad(x, ((0, _NP - _N), (0, 0)))
  srcp = jnp.pad(src, (0, _EP - _E), constant_values=_NP - 1)
  dstp = jnp.pad(dst, (0, _EP - _E), constant_values=_NP - 1)
  gidx = jnp.pad(graph_indices, (0, _NP - _N),
                 constant_values=_G).reshape(_NP, 1)
  b1r = b1.reshape(1, _D)
  b2r = b2.reshape(1, _D)
  bpr = bp.reshape(1, 1)
  bvr = bv.reshape(1, 1)

  parts1, deg = _sc_agg_deg(xp, srcp, dstp)
  degr = deg.reshape(_NC * _NS, 1, _NP)
  h1, recip = _tc1(parts1, degr, xp, W1l, W1r, b1r)
  (parts2,) = _sc_agg(h1, srcp, dstp)
  pl_lane, pr2d, gp = _tc2(parts2, recip, h1, W2l, W2r, b2r, Wpl, Wpr, gidx)
  (pparts3,) = _sc_pol_local(pl_lane.reshape(_NP), srcp, dstp)
  pp4 = pparts3.reshape(_NC, _NS, 1, _NP)
  policy, value = _tc3(pp4, recip, pr2d, bpr, gp, Wv, bvr)
  return (policy[:_N], value)
